# baseline (speedup 1.0000x reference)
import functools
import jax
import jax.numpy as jnp
from jax import lax
from jax.experimental import pallas as pl
from jax.experimental.pallas import tpu as pltpu
from jax.experimental.pallas import tpu_sc as plsc


def kernel(x, person_emb):
    B, T, P, D = x.shape
    R = 10
    NCH = T // R

    info = plsc.get_sparse_core_info()
    NC, NS = info.num_cores, info.num_subcores
    assert NC * NS == B

    mesh = plsc.VectorSubcoreMesh(core_axis_name="c", subcore_axis_name="s")

    @functools.partial(
        pl.kernel,
        mesh=mesh,
        out_type=jax.ShapeDtypeStruct((B, T, P, D), person_emb.dtype),
        scratch_types=[
            pltpu.VMEM((R, P, D), person_emb.dtype),
            pltpu.SemaphoreType.DMA,
        ],
    )
    def k(emb_hbm, out_hbm, buf, sem):
        w = lax.axis_index("s") * NC + lax.axis_index("c")
        fills = [pltpu.async_copy(emb_hbm, buf.at[i], sem) for i in range(R)]
        for f in fills:
            f.wait()
        streams = [
            pltpu.async_copy(buf, out_hbm.at[w, pl.ds(j * R, R)], sem)
            for j in range(NCH)
        ]
        for s in streams:
            s.wait()

    return k(person_emb)


# speedup vs baseline: 1.1835x; 1.1835x over previous
import jax
import jax.numpy as jnp
from jax.experimental import pallas as pl
from jax.experimental.pallas import tpu as pltpu


def kernel(x, person_emb):
    B, T, P, D = x.shape
    T_BLK = 100
    NJ = T // T_BLK

    NSEM = 8

    def body(emb_ref, o_ref, buf, sems):
        buf[...] = jnp.broadcast_to(emb_ref[...][None, :, :], (T_BLK, P, D))
        k = 0
        for i in range(B):
            for j in range(NJ):
                pltpu.make_async_copy(
                    buf, o_ref.at[i, pl.ds(j * T_BLK, T_BLK)], sems.at[k % NSEM]
                ).start()
                k += 1
        for k in range(B * NJ):
            pltpu.make_async_copy(
                buf, o_ref.at[0, pl.ds(0, T_BLK)], sems.at[k % NSEM]
            ).wait()

    return pl.pallas_call(
        body,
        in_specs=[pl.BlockSpec(memory_space=pltpu.VMEM)],
        out_specs=pl.BlockSpec(memory_space=pl.ANY),
        out_shape=jax.ShapeDtypeStruct((B, T, P, D), person_emb.dtype),
        scratch_shapes=[
            pltpu.VMEM((T_BLK, P, D), person_emb.dtype),
            pltpu.SemaphoreType.DMA((NSEM,)),
        ],
    )(person_emb)
